# baseline (speedup 1.0000x reference)
import functools

import jax
import jax.numpy as jnp
from jax import lax
from jax.experimental import pallas as pl
from jax.experimental.pallas import tpu as pltpu
from jax.experimental.pallas import tpu_sc as plsc

N = 10000
NP = 10240
E = 320000
EP = E + 128
HID = 128
HD2 = 144
OUT = 3
NC = 2
NS = 16
NW = NC * NS
L = 16
EW = E // NW
C = 80
NCH = EW // C
G = C // L
RPT = NP // NW // L
ZR = 32

_mesh = plsc.VectorSubcoreMesh(
    core_axis_name="c", subcore_axis_name="s", num_cores=NC, num_subcores=NS)
_params = pltpu.CompilerParams(needs_layout_passes=False)

_f32 = jnp.float32


def _wid():
    return lax.axis_index("c") * NS + lax.axis_index("s")


@functools.partial(
    pl.kernel,
    out_type=(jax.ShapeDtypeStruct((EP,), _f32),
              jax.ShapeDtypeStruct((NW * L,), _f32)),
    mesh=_mesh,
    compiler_params=_params,
    scratch_types=[
        pltpu.VMEM((NP,), _f32),
        pltpu.VMEM((NP,), _f32),
        pltpu.VMEM((NP,), _f32),
        pltpu.VMEM((HID,), _f32),
        pltpu.VMEM((HID,), _f32),
        pltpu.VMEM((HID,), _f32),
        pltpu.VMEM((HID,), _f32),
        pltpu.VMEM((2, C), jnp.int32),
        pltpu.VMEM((2, C), jnp.int32),
        pltpu.VMEM((2, C, HID), _f32),
        pltpu.VMEM((2, C, HID), _f32),
        pltpu.VMEM((2, C), _f32),
        pltpu.VMEM((L,), _f32),
        pltpu.SemaphoreType.DMA,
        pltpu.SemaphoreType.DMA,
        pltpu.SemaphoreType.DMA,
        pltpu.SemaphoreType.DMA,
        pltpu.SemaphoreType.DMA,
        pltpu.SemaphoreType.DMA,
    ])
def _sc_attn(xl, xr, px, py, pz, srcp, dstp, att_h, wex_h, wey_h, wez_h,
             alpha_o, pmax_o,
             px_v, py_v, pz_v, att_v, wex_v, wey_v, wez_v,
             se_v, de_v, xj_v, xi_v, al_v, mx_v,
             s_ei0, s_ei1, s_gat0, s_gat1, s_al0, s_al1):
    wid = _wid()
    base_e = wid * EW
    s_ei = (s_ei0, s_ei1)
    s_gat = (s_gat0, s_gat1)
    s_al = (s_al0, s_al1)
    pltpu.sync_copy(px, px_v)
    pltpu.sync_copy(py, py_v)
    pltpu.sync_copy(pz, pz_v)
    pltpu.sync_copy(att_h, att_v)
    pltpu.sync_copy(wex_h, wex_v)
    pltpu.sync_copy(wey_h, wey_v)
    pltpu.sync_copy(wez_h, wez_v)
    iot = lax.iota(jnp.int32, L)
    colv = [c * L + iot for c in range(HID // L)]
    lmask = [iot == lane for lane in range(L)]
    wxc = [wex_v[pl.ds(c * L, L)] for c in range(HID // L)]
    wyc = [wey_v[pl.ds(c * L, L)] for c in range(HID // L)]
    wzc = [wez_v[pl.ds(c * L, L)] for c in range(HID // L)]
    afc = [att_v[pl.ds(c * L, L)] for c in range(HID // L)]

    def start_ei(j, b):
        pltpu.async_copy(srcp.at[pl.ds(base_e + j * C, C)], se_v.at[b],
                         s_ei[b])
        pltpu.async_copy(dstp.at[pl.ds(base_e + j * C, C)], de_v.at[b],
                         s_ei[b])

    def wait_ei(b):
        pltpu.make_async_copy(srcp.at[pl.ds(base_e, C)], se_v.at[b],
                              s_ei[b]).wait()
        pltpu.make_async_copy(dstp.at[pl.ds(base_e, C)], de_v.at[b],
                              s_ei[b]).wait()

    def start_gat(j, b):
        del j
        pltpu.async_copy(xl.at[se_v.at[b]], xj_v.at[b], s_gat[b])
        pltpu.async_copy(xr.at[de_v.at[b]], xi_v.at[b], s_gat[b])

    def wait_gat(b):
        pltpu.make_async_copy(xl.at[se_v.at[b]], xj_v.at[b],
                              s_gat[b]).wait()
        pltpu.make_async_copy(xr.at[de_v.at[b]], xi_v.at[b],
                              s_gat[b]).wait()

    def wait_al(b):
        pltpu.make_async_copy(al_v.at[b], alpha_o.at[pl.ds(base_e, C)],
                              s_al[b]).wait()

    def process(j, b, maxv, last):
        b1 = 1 - b

        @pl.when(j >= 2)
        def _():
            wait_al(b)

        wait_gat(b)
        if not last:
            wait_ei(b1)
            start_gat(j + 1, b1)

        def g_step(g, maxv):
            gs = pl.ds(g * L, L)
            sg = se_v[b, gs]
            dg = de_v[b, gs]
            exv = plsc.load_gather(px_v, [dg]) - plsc.load_gather(px_v, [sg])
            eyv = plsc.load_gather(py_v, [dg]) - plsc.load_gather(py_v, [sg])
            ezv = plsc.load_gather(pz_v, [dg]) - plsc.load_gather(pz_v, [sg])
            alpha_g = jnp.zeros((L,), _f32)
            for lane in range(L):
                rowv = jnp.full((L,), 0, jnp.int32) + (g * L + lane)
                ex_s = exv[lane]
                ey_s = eyv[lane]
                ez_s = ezv[lane]
                acc = jnp.zeros((L,), _f32)
                for c in range(HID // L):
                    xj = plsc.load_gather(xj_v.at[b], [rowv, colv[c]])
                    xi = plsc.load_gather(xi_v.at[b], [rowv, colv[c]])
                    v = (xj + xi + wxc[c] * ex_s + wyc[c] * ey_s
                         + wzc[c] * ez_s)
                    acc = acc + afc[c] * jnp.maximum(v, 0.2 * v)
                alpha_g = jnp.where(lmask[lane], jnp.sum(acc), alpha_g)
            al_v[b, gs] = alpha_g
            return jnp.maximum(maxv, alpha_g)

        maxv = lax.fori_loop(0, G, g_step, maxv)
        if not last:
            start_ei(j + 2, b)
        pltpu.async_copy(al_v.at[b], alpha_o.at[pl.ds(base_e + j * C, C)],
                         s_al[b])
        return maxv

    start_ei(0, 0)
    wait_ei(0)
    start_gat(0, 0)
    start_ei(1, 1)

    def pair(i, maxv):
        j0 = i * 2
        maxv = process(j0, 0, maxv, False)
        maxv = process(j0 + 1, 1, maxv, False)
        return maxv

    maxv = lax.fori_loop(0, NCH // 2, pair,
                         jnp.full((L,), -3.4e38, _f32))
    maxv = process(NCH - 1, 0, maxv, True)
    wait_al(1)
    wait_al(0)
    wait_ei(1)
    mx_v[...] = maxv
    pltpu.sync_copy(mx_v, pmax_o.at[pl.ds(wid * L, L)])


@functools.partial(
    pl.kernel,
    out_type=(jax.ShapeDtypeStruct((NC, NP, HID), _f32),
              jax.ShapeDtypeStruct((NC, NP), _f32)),
    mesh=_mesh,
    compiler_params=_params,
    scratch_types=[
        pltpu.VMEM_SHARED((NP, HID), _f32),
        pltpu.VMEM_SHARED((NP,), _f32),
        pltpu.VMEM((NW * L,), _f32),
        pltpu.VMEM((2, C), jnp.int32),
        pltpu.VMEM((2, C), jnp.int32),
        pltpu.VMEM((2, C), jnp.int32),
        pltpu.VMEM((2, C, HID), _f32),
        pltpu.VMEM((2, C, HID), _f32),
        pltpu.VMEM((2, C), _f32),
        pltpu.VMEM((2, C), _f32),
        pltpu.VMEM((ZR, HID), _f32),
        pltpu.VMEM((NP // NS,), _f32),
        pltpu.SemaphoreType.DMA,
        pltpu.SemaphoreType.DMA,
        pltpu.SemaphoreType.DMA,
        pltpu.SemaphoreType.DMA,
        pltpu.SemaphoreType.DMA,
        pltpu.SemaphoreType.DMA,
        pltpu.SemaphoreType.DMA,
        pltpu.SemaphoreType.DMA,
        pltpu.SemaphoreType.DMA,
        pltpu.SemaphoreType.DMA,
    ])
def _sc_aggr(xl, srcp, dstp, alpha_i, pmax_i,
             outp_o, denp_o,
             out_sh, den_sh, pm_v, se_v, de_v, dsti_v, xjg_v, sc_v, ali_v,
             ex_v, z_v, dz_v,
             s_ei0, s_ei1, s_ali0, s_ali1, s_gat0, s_gat1, s_sc0, s_sc1,
             s_scx0, s_scx1):
    cid = lax.axis_index("c")
    sid = lax.axis_index("s")
    wid = cid * NS + sid
    base_e = wid * EW
    s_ei = (s_ei0, s_ei1)
    s_ali = (s_ali0, s_ali1)
    s_gat = (s_gat0, s_gat1)
    s_sc = (s_sc0, s_sc1)
    s_scx = (s_scx0, s_scx1)
    iot = lax.iota(jnp.int32, L)
    colv = [c * L + iot for c in range(HID // L)]
    zv = jnp.zeros((L,), _f32)
    for r in range(ZR):
        for c8 in range(HID // L):
            z_v[r, pl.ds(c8 * L, L)] = zv
    nds = NP // NS

    def dz_fill(i, _):
        dz_v[pl.ds(i * L, L)] = zv
        return 0

    lax.fori_loop(0, nds // L, dz_fill, 0)
    rb = sid * nds
    for t in range(nds // ZR):
        pltpu.sync_copy(z_v, out_sh.at[pl.ds(rb + t * ZR, ZR), :])
    pltpu.sync_copy(dz_v, den_sh.at[pl.ds(rb, nds)])
    plsc.subcore_barrier()

    pltpu.sync_copy(pmax_i, pm_v)
    mv = pm_v[pl.ds(0, L)]
    for i in range(1, NW):
        mv = jnp.maximum(mv, pm_v[pl.ds(i * L, L)])
    m_glob = jnp.max(mv)

    def start_ei(j, b):
        pltpu.async_copy(srcp.at[pl.ds(base_e + j * C, C)], se_v.at[b],
                         s_ei[b])
        pltpu.async_copy(dstp.at[pl.ds(base_e + j * C, C)], de_v.at[b],
                         s_ei[b])
        pltpu.async_copy(alpha_i.at[pl.ds(base_e + j * C, C)], ali_v.at[b],
                         s_ali[b])

    def wait_ei(b):
        pltpu.make_async_copy(srcp.at[pl.ds(base_e, C)], se_v.at[b],
                              s_ei[b]).wait()
        pltpu.make_async_copy(dstp.at[pl.ds(base_e, C)], de_v.at[b],
                              s_ei[b]).wait()
        pltpu.make_async_copy(alpha_i.at[pl.ds(base_e, C)], ali_v.at[b],
                              s_ali[b]).wait()

    def start_gat(j, b):
        del j
        pltpu.async_copy(xl.at[se_v.at[b]], xjg_v.at[b], s_gat[b])

    def wait_gat(b):
        pltpu.make_async_copy(xl.at[se_v.at[b]], xjg_v.at[b],
                              s_gat[b]).wait()

    def wait_sc(b):
        pltpu.make_async_copy(sc_v.at[b], out_sh.at[dsti_v.at[b]],
                              s_sc[b]).wait()
        pltpu.make_async_copy(ex_v.at[b], den_sh.at[dsti_v.at[b]],
                              s_scx[b]).wait()

    def process(j, b, last):
        b1 = 1 - b

        @pl.when(j >= 2)
        def _():
            wait_sc(b)

        wait_gat(b)
        if not last:
            wait_ei(b1)
            start_gat(j + 1, b1)
        exgs = []
        for g in range(G):
            gs = pl.ds(g * L, L)
            dsti_v[b, gs] = de_v[b, gs]
            eg = jnp.exp(ali_v[b, gs] - m_glob)
            exgs.append(eg)
            ex_v[b, gs] = eg
        if not last:
            start_ei(j + 2, b)
        pltpu.async_copy(ex_v.at[b], den_sh.at[dsti_v.at[b]], s_scx[b],
                         add=True)

        def g_step(g, carry):
            exv = ex_v[b, pl.ds(g * L, L)]
            for lane in range(L):
                rowv = jnp.full((L,), 0, jnp.int32) + (g * L + lane)
                ex_s = exv[lane]
                for c in range(HID // L):
                    xv = plsc.load_gather(xjg_v.at[b], [rowv, colv[c]])
                    plsc.store_scatter(sc_v.at[b], [rowv, colv[c]],
                                       xv * ex_s)
            return carry

        lax.fori_loop(0, G, g_step, 0)
        pltpu.async_copy(sc_v.at[b], out_sh.at[dsti_v.at[b]], s_sc[b],
                         add=True)

    start_ei(0, 0)
    wait_ei(0)
    start_gat(0, 0)
    start_ei(1, 1)

    def pair(i, _):
        j0 = i * 2
        process(j0, 0, False)
        process(j0 + 1, 1, False)
        return 0

    lax.fori_loop(0, NCH // 2, pair, 0)
    process(NCH - 1, 0, True)
    wait_sc(1)
    wait_sc(0)
    wait_ei(1)
    plsc.subcore_barrier()
    for t in range(nds // ZR):
        pltpu.sync_copy(out_sh.at[pl.ds(rb + t * ZR, ZR), :],
                        outp_o.at[cid, pl.ds(rb + t * ZR, ZR), :])
    pltpu.sync_copy(den_sh.at[pl.ds(rb, nds)], denp_o.at[cid, pl.ds(rb, nds)])


DR = 64
RPD = NP // NW // DR
@functools.partial(
    pl.kernel,
    out_type=jax.ShapeDtypeStruct((NP, HID), _f32),
    mesh=_mesh,
    compiler_params=_params,
    scratch_types=[
        pltpu.VMEM((HID,), _f32),
        pltpu.VMEM((DR, HID), _f32),
        pltpu.VMEM((DR, HID), _f32),
        pltpu.VMEM((DR,), _f32),
        pltpu.VMEM((DR,), _f32),
        pltpu.VMEM((DR, HID), _f32),
        pltpu.VMEM((DR, HID), _f32),
    ])
def _sc_norm(outp_i, denp_i, bias_h, skip_i, x_o,
             b_v, o0_v, o1_v, d0_v, d1_v, sk_v, xb_v):
    wid = _wid()
    pltpu.sync_copy(bias_h, b_v)
    bcs = [b_v[pl.ds(c8 * L, L)] for c8 in range(HID // L)]

    def row_chunk(k, _):
        rb = (wid * RPD + k) * DR
        pltpu.sync_copy(outp_i.at[0, pl.ds(rb, DR), :], o0_v)
        pltpu.sync_copy(outp_i.at[1, pl.ds(rb, DR), :], o1_v)
        pltpu.sync_copy(denp_i.at[0, pl.ds(rb, DR)], d0_v)
        pltpu.sync_copy(denp_i.at[1, pl.ds(rb, DR)], d1_v)
        pltpu.sync_copy(skip_i.at[pl.ds(rb, DR), :], sk_v)
        for rg in range(DR // L):
            d0c = d0_v[pl.ds(rg * L, L)]
            d1c = d1_v[pl.ds(rg * L, L)]
            dall = d0c + d1c + 1e-16
            for rl in range(L):
                r = rg * L + rl
                d = dall[rl]
                for c8 in range(HID // L):
                    cs = pl.ds(c8 * L, L)
                    v = (o0_v[r, cs] + o1_v[r, cs]) / d + bcs[c8] + sk_v[r, cs]
                    xb_v[r, cs] = jnp.where(v > 0.0, v, jnp.exp(v) - 1.0)
        pltpu.sync_copy(xb_v, x_o.at[pl.ds(rb, DR), :])
        return 0

    lax.fori_loop(0, RPD, row_chunk, 0)


def _mm_body(nw):
    def body(x_ref, p_ref, wx_ref, wp_ref, b_ref, *out_refs):
        x = x_ref[...]
        p = p_ref[...]
        for i in range(nw):
            acc = jnp.dot(x, wx_ref[i], preferred_element_type=_f32)
            acc = acc + jnp.dot(p, wp_ref[i], preferred_element_type=_f32)
            out_refs[i][...] = acc + b_ref[i, 0:1, :]
    return body


def _lin(x, posp, wxs, wps, bs, nw):
    bn = 2048
    return pl.pallas_call(
        _mm_body(nw),
        grid=(NP // bn,),
        in_specs=[
            pl.BlockSpec((bn, HID), lambda i: (i, 0)),
            pl.BlockSpec((bn, HID), lambda i: (i, 0)),
            pl.BlockSpec((nw, HID, HID), lambda i: (0, 0, 0)),
            pl.BlockSpec((nw, HID, HID), lambda i: (0, 0, 0)),
            pl.BlockSpec((nw, 8, HID), lambda i: (0, 0, 0)),
        ],
        out_specs=[pl.BlockSpec((bn, HID), lambda i: (i, 0))] * nw,
        out_shape=[jax.ShapeDtypeStruct((NP, HID), _f32)] * nw,
    )(x, posp, wxs, wps, bs)


def _pad_posw(w):
    return jnp.zeros((HID, HID), _f32).at[:3, :].set(w[:, HID:].T)


def _pad_bias(b):
    return jnp.zeros((8, HID), _f32).at[0, :b.shape[0]].set(b)


def kernel(latent, pos, edge_attr, params, edge_index):
    del edge_attr
    prm = params
    src_p = jnp.pad(edge_index[0], (0, EP - E))
    dst_p = jnp.pad(edge_index[1], (0, EP - E))
    lat_p = jnp.pad(latent.astype(_f32), ((0, NP - N), (0, 0)))
    pos_p = jnp.pad(pos.astype(_f32), ((0, NP - N), (0, HID - 3)))
    px = jnp.pad(pos[:, 0], (0, NP - N))
    py = jnp.pad(pos[:, 1], (0, NP - N))
    pz = jnp.pad(pos[:, 2], (0, NP - N))

    zero_w = jnp.zeros((1, HID, HID), _f32)
    x = _lin(lat_p, pos_p, prm['W0'].T[None], zero_w,
             _pad_bias(prm['b0'])[None], 1)[0]

    layers = [(prm['gat0'], prm['W1'], prm['b1']),
              (prm['gat1'], prm['W2'], prm['b2']),
              (prm['gat2'], prm['W3'], prm['b3'])]
    for gat, wk, bk in layers:
        wxs = jnp.stack([gat['Wl'][:, :HID].T, gat['Wr'][:, :HID].T,
                         wk[:, :HID].T])
        wps = jnp.stack([_pad_posw(gat['Wl']), _pad_posw(gat['Wr']),
                         _pad_posw(wk)])
        bss = jnp.stack([_pad_bias(gat['bl']), _pad_bias(gat['br']),
                         _pad_bias(bk)])
        xl, xr, skip = _lin(x, pos_p, wxs, wps, bss, 3)
        alpha, pmax = _sc_attn(xl, xr, px, py, pz, src_p, dst_p,
                               gat['att'], gat['We'][:, 0],
                               gat['We'][:, 1], gat['We'][:, 2])
        outp, denp = _sc_aggr(xl, src_p, dst_p, alpha, pmax)
        x = _sc_norm(outp, denp, gat['bias'], skip)

    w4x = jnp.zeros((HID, HID), _f32).at[:, :OUT].set(prm['W4'][:, :HID].T)
    w4p = jnp.zeros((HID, HID), _f32).at[:3, :OUT].set(prm['W4'][:, HID:].T)
    out = _lin(x, pos_p, w4x[None], w4p[None], _pad_bias(prm['b4'])[None],
               1)[0]
    return out[:N, :OUT]


# speedup vs baseline: 1.0745x; 1.0745x over previous
import functools

import jax
import jax.numpy as jnp
from jax import lax
from jax.experimental import pallas as pl
from jax.experimental.pallas import tpu as pltpu
from jax.experimental.pallas import tpu_sc as plsc

N = 10000
NP = 10240
E = 320000
EP = E + 128
HID = 128
HD2 = 144
OUT = 3
NC = 2
NS = 16
NW = NC * NS
L = 16
EW = E // NW
C = 80
NCH = EW // C
G = C // L
RPT = NP // NW // L
ZR = 32

_mesh = plsc.VectorSubcoreMesh(
    core_axis_name="c", subcore_axis_name="s", num_cores=NC, num_subcores=NS)
_params = pltpu.CompilerParams(needs_layout_passes=False)

_f32 = jnp.float32


def _wid():
    return lax.axis_index("c") * NS + lax.axis_index("s")


@functools.partial(
    pl.kernel,
    out_type=(jax.ShapeDtypeStruct((EP,), _f32),
              jax.ShapeDtypeStruct((NW * L,), _f32)),
    mesh=_mesh,
    compiler_params=_params,
    scratch_types=[
        pltpu.VMEM((NP,), _f32),
        pltpu.VMEM((NP,), _f32),
        pltpu.VMEM((NP,), _f32),
        pltpu.VMEM((HID,), _f32),
        pltpu.VMEM((HID,), _f32),
        pltpu.VMEM((HID,), _f32),
        pltpu.VMEM((HID,), _f32),
        pltpu.VMEM((2, C), jnp.int32),
        pltpu.VMEM((2, C), jnp.int32),
        pltpu.VMEM((2, C, HID), _f32),
        pltpu.VMEM((2, C, HID), _f32),
        pltpu.VMEM((2, C), _f32),
        pltpu.VMEM((L,), _f32),
        pltpu.SemaphoreType.DMA,
        pltpu.SemaphoreType.DMA,
        pltpu.SemaphoreType.DMA,
        pltpu.SemaphoreType.DMA,
        pltpu.SemaphoreType.DMA,
        pltpu.SemaphoreType.DMA,
    ])
def _sc_attn(xl, xr, px, py, pz, srcp, dstp, att_h, wex_h, wey_h, wez_h,
             alpha_o, pmax_o,
             px_v, py_v, pz_v, att_v, wex_v, wey_v, wez_v,
             se_v, de_v, xj_v, xi_v, al_v, mx_v,
             s_ei0, s_ei1, s_gat0, s_gat1, s_al0, s_al1):
    wid = _wid()
    base_e = wid * EW
    s_ei = (s_ei0, s_ei1)
    s_gat = (s_gat0, s_gat1)
    s_al = (s_al0, s_al1)
    pltpu.sync_copy(px, px_v)
    pltpu.sync_copy(py, py_v)
    pltpu.sync_copy(pz, pz_v)
    pltpu.sync_copy(att_h, att_v)
    pltpu.sync_copy(wex_h, wex_v)
    pltpu.sync_copy(wey_h, wey_v)
    pltpu.sync_copy(wez_h, wez_v)
    iot = lax.iota(jnp.int32, L)
    colv = [c * L + iot for c in range(HID // L)]
    lmask = [iot == lane for lane in range(L)]

    def start_ei(j, b):
        pltpu.async_copy(srcp.at[pl.ds(base_e + j * C, C)], se_v.at[b],
                         s_ei[b])
        pltpu.async_copy(dstp.at[pl.ds(base_e + j * C, C)], de_v.at[b],
                         s_ei[b])

    def wait_ei(b):
        pltpu.make_async_copy(srcp.at[pl.ds(base_e, C)], se_v.at[b],
                              s_ei[b]).wait()
        pltpu.make_async_copy(dstp.at[pl.ds(base_e, C)], de_v.at[b],
                              s_ei[b]).wait()

    def start_gat(j, b):
        del j
        pltpu.async_copy(xl.at[se_v.at[b]], xj_v.at[b], s_gat[b])
        pltpu.async_copy(xr.at[de_v.at[b]], xi_v.at[b], s_gat[b])

    def wait_gat(b):
        pltpu.make_async_copy(xl.at[se_v.at[b]], xj_v.at[b],
                              s_gat[b]).wait()
        pltpu.make_async_copy(xr.at[de_v.at[b]], xi_v.at[b],
                              s_gat[b]).wait()

    def wait_al(b):
        pltpu.make_async_copy(al_v.at[b], alpha_o.at[pl.ds(base_e, C)],
                              s_al[b]).wait()

    def process(j, b, maxv, last):
        b1 = 1 - b

        @pl.when(j >= 2)
        def _():
            wait_al(b)

        wait_gat(b)
        if not last:
            wait_ei(b1)
            start_gat(j + 1, b1)

        wxc = [wex_v[pl.ds(c * L, L)] for c in range(HID // L)]
        wyc = [wey_v[pl.ds(c * L, L)] for c in range(HID // L)]
        wzc = [wez_v[pl.ds(c * L, L)] for c in range(HID // L)]
        afc = [att_v[pl.ds(c * L, L)] for c in range(HID // L)]

        def g_step(g, maxv):
            gs = pl.ds(g * L, L)
            sg = se_v[b, gs]
            dg = de_v[b, gs]
            exv = plsc.load_gather(px_v, [dg]) - plsc.load_gather(px_v, [sg])
            eyv = plsc.load_gather(py_v, [dg]) - plsc.load_gather(py_v, [sg])
            ezv = plsc.load_gather(pz_v, [dg]) - plsc.load_gather(pz_v, [sg])
            alpha_g = jnp.zeros((L,), _f32)
            for lane in range(L):
                rowv = jnp.full((L,), 0, jnp.int32) + (g * L + lane)
                ex_s = exv[lane]
                ey_s = eyv[lane]
                ez_s = ezv[lane]
                acc = jnp.zeros((L,), _f32)
                for c in range(HID // L):
                    xj = plsc.load_gather(xj_v.at[b], [rowv, colv[c]])
                    xi = plsc.load_gather(xi_v.at[b], [rowv, colv[c]])
                    v = (xj + xi + wxc[c] * ex_s + wyc[c] * ey_s
                         + wzc[c] * ez_s)
                    acc = acc + afc[c] * jnp.maximum(v, 0.2 * v)
                alpha_g = jnp.where(lmask[lane], jnp.sum(acc), alpha_g)
            al_v[b, gs] = alpha_g
            return jnp.maximum(maxv, alpha_g)

        maxv = lax.fori_loop(0, G, g_step, maxv)
        if not last:
            start_ei(j + 2, b)
        pltpu.async_copy(al_v.at[b], alpha_o.at[pl.ds(base_e + j * C, C)],
                         s_al[b])
        return maxv

    start_ei(0, 0)
    wait_ei(0)
    start_gat(0, 0)
    start_ei(1, 1)

    def pair(i, maxv):
        j0 = i * 2
        maxv = process(j0, 0, maxv, False)
        maxv = process(j0 + 1, 1, maxv, False)
        return maxv

    maxv = lax.fori_loop(0, NCH // 2, pair,
                         jnp.full((L,), -3.4e38, _f32))
    maxv = process(NCH - 1, 0, maxv, True)
    wait_al(1)
    wait_al(0)
    wait_ei(1)
    mx_v[...] = maxv
    pltpu.sync_copy(mx_v, pmax_o.at[pl.ds(wid * L, L)])


@functools.partial(
    pl.kernel,
    out_type=(jax.ShapeDtypeStruct((NC, NP, HID), _f32),
              jax.ShapeDtypeStruct((NC, NP), _f32)),
    mesh=_mesh,
    compiler_params=_params,
    scratch_types=[
        pltpu.VMEM_SHARED((NP, HID), _f32),
        pltpu.VMEM_SHARED((NP,), _f32),
        pltpu.VMEM((NW * L,), _f32),
        pltpu.VMEM((2, C), jnp.int32),
        pltpu.VMEM((2, C), jnp.int32),
        pltpu.VMEM((2, C), jnp.int32),
        pltpu.VMEM((2, C, HID), _f32),
        pltpu.VMEM((2, C, HID), _f32),
        pltpu.VMEM((2, C), _f32),
        pltpu.VMEM((2, C), _f32),
        pltpu.VMEM((ZR, HID), _f32),
        pltpu.VMEM((NP // NS,), _f32),
        pltpu.SemaphoreType.DMA,
        pltpu.SemaphoreType.DMA,
        pltpu.SemaphoreType.DMA,
        pltpu.SemaphoreType.DMA,
        pltpu.SemaphoreType.DMA,
        pltpu.SemaphoreType.DMA,
        pltpu.SemaphoreType.DMA,
        pltpu.SemaphoreType.DMA,
        pltpu.SemaphoreType.DMA,
        pltpu.SemaphoreType.DMA,
    ])
def _sc_aggr(xl, srcp, dstp, alpha_i, pmax_i,
             outp_o, denp_o,
             out_sh, den_sh, pm_v, se_v, de_v, dsti_v, xjg_v, sc_v, ali_v,
             ex_v, z_v, dz_v,
             s_ei0, s_ei1, s_ali0, s_ali1, s_gat0, s_gat1, s_sc0, s_sc1,
             s_scx0, s_scx1):
    cid = lax.axis_index("c")
    sid = lax.axis_index("s")
    wid = cid * NS + sid
    base_e = wid * EW
    s_ei = (s_ei0, s_ei1)
    s_ali = (s_ali0, s_ali1)
    s_gat = (s_gat0, s_gat1)
    s_sc = (s_sc0, s_sc1)
    s_scx = (s_scx0, s_scx1)
    iot = lax.iota(jnp.int32, L)
    colv = [c * L + iot for c in range(HID // L)]
    zv = jnp.zeros((L,), _f32)
    for r in range(ZR):
        for c8 in range(HID // L):
            z_v[r, pl.ds(c8 * L, L)] = zv
    nds = NP // NS

    def dz_fill(i, _):
        dz_v[pl.ds(i * L, L)] = zv
        return 0

    lax.fori_loop(0, nds // L, dz_fill, 0)
    rb = sid * nds
    for t in range(nds // ZR):
        pltpu.sync_copy(z_v, out_sh.at[pl.ds(rb + t * ZR, ZR), :])
    pltpu.sync_copy(dz_v, den_sh.at[pl.ds(rb, nds)])
    plsc.subcore_barrier()

    pltpu.sync_copy(pmax_i, pm_v)
    mv = pm_v[pl.ds(0, L)]
    for i in range(1, NW):
        mv = jnp.maximum(mv, pm_v[pl.ds(i * L, L)])
    m_glob = jnp.max(mv)

    def start_ei(j, b):
        pltpu.async_copy(srcp.at[pl.ds(base_e + j * C, C)], se_v.at[b],
                         s_ei[b])
        pltpu.async_copy(dstp.at[pl.ds(base_e + j * C, C)], de_v.at[b],
                         s_ei[b])
        pltpu.async_copy(alpha_i.at[pl.ds(base_e + j * C, C)], ali_v.at[b],
                         s_ali[b])

    def wait_ei(b):
        pltpu.make_async_copy(srcp.at[pl.ds(base_e, C)], se_v.at[b],
                              s_ei[b]).wait()
        pltpu.make_async_copy(dstp.at[pl.ds(base_e, C)], de_v.at[b],
                              s_ei[b]).wait()
        pltpu.make_async_copy(alpha_i.at[pl.ds(base_e, C)], ali_v.at[b],
                              s_ali[b]).wait()

    def start_gat(j, b):
        del j
        pltpu.async_copy(xl.at[se_v.at[b]], xjg_v.at[b], s_gat[b])

    def wait_gat(b):
        pltpu.make_async_copy(xl.at[se_v.at[b]], xjg_v.at[b],
                              s_gat[b]).wait()

    def wait_sc(b):
        pltpu.make_async_copy(sc_v.at[b], out_sh.at[dsti_v.at[b]],
                              s_sc[b]).wait()
        pltpu.make_async_copy(ex_v.at[b], den_sh.at[dsti_v.at[b]],
                              s_scx[b]).wait()

    def process(j, b, last):
        b1 = 1 - b

        @pl.when(j >= 2)
        def _():
            wait_sc(b)

        wait_gat(b)
        if not last:
            wait_ei(b1)
            start_gat(j + 1, b1)
        exgs = []
        for g in range(G):
            gs = pl.ds(g * L, L)
            dsti_v[b, gs] = de_v[b, gs]
            eg = jnp.exp(ali_v[b, gs] - m_glob)
            exgs.append(eg)
            ex_v[b, gs] = eg
        if not last:
            start_ei(j + 2, b)
        pltpu.async_copy(ex_v.at[b], den_sh.at[dsti_v.at[b]], s_scx[b],
                         add=True)

        def g_step(g, carry):
            exv = ex_v[b, pl.ds(g * L, L)]
            for lane in range(L):
                rowv = jnp.full((L,), 0, jnp.int32) + (g * L + lane)
                ex_s = exv[lane]
                for c in range(HID // L):
                    xv = plsc.load_gather(xjg_v.at[b], [rowv, colv[c]])
                    plsc.store_scatter(sc_v.at[b], [rowv, colv[c]],
                                       xv * ex_s)
            return carry

        lax.fori_loop(0, G, g_step, 0)
        pltpu.async_copy(sc_v.at[b], out_sh.at[dsti_v.at[b]], s_sc[b],
                         add=True)

    start_ei(0, 0)
    wait_ei(0)
    start_gat(0, 0)
    start_ei(1, 1)

    def pair(i, _):
        j0 = i * 2
        process(j0, 0, False)
        process(j0 + 1, 1, False)
        return 0

    lax.fori_loop(0, NCH // 2, pair, 0)
    process(NCH - 1, 0, True)
    wait_sc(1)
    wait_sc(0)
    wait_ei(1)
    plsc.subcore_barrier()
    for t in range(nds // ZR):
        pltpu.sync_copy(out_sh.at[pl.ds(rb + t * ZR, ZR), :],
                        outp_o.at[cid, pl.ds(rb + t * ZR, ZR), :])
    pltpu.sync_copy(den_sh.at[pl.ds(rb, nds)], denp_o.at[cid, pl.ds(rb, nds)])


DR = 64
RPD = NP // NW // DR
@functools.partial(
    pl.kernel,
    out_type=jax.ShapeDtypeStruct((NP, HID), _f32),
    mesh=_mesh,
    compiler_params=_params,
    scratch_types=[
        pltpu.VMEM((HID,), _f32),
        pltpu.VMEM((DR, HID), _f32),
        pltpu.VMEM((DR, HID), _f32),
        pltpu.VMEM((DR,), _f32),
        pltpu.VMEM((DR,), _f32),
        pltpu.VMEM((DR, HID), _f32),
        pltpu.VMEM((DR, HID), _f32),
    ])
def _sc_norm(outp_i, denp_i, bias_h, skip_i, x_o,
             b_v, o0_v, o1_v, d0_v, d1_v, sk_v, xb_v):
    wid = _wid()
    pltpu.sync_copy(bias_h, b_v)
    bcs = [b_v[pl.ds(c8 * L, L)] for c8 in range(HID // L)]

    def row_chunk(k, _):
        rb = (wid * RPD + k) * DR
        pltpu.sync_copy(outp_i.at[0, pl.ds(rb, DR), :], o0_v)
        pltpu.sync_copy(outp_i.at[1, pl.ds(rb, DR), :], o1_v)
        pltpu.sync_copy(denp_i.at[0, pl.ds(rb, DR)], d0_v)
        pltpu.sync_copy(denp_i.at[1, pl.ds(rb, DR)], d1_v)
        pltpu.sync_copy(skip_i.at[pl.ds(rb, DR), :], sk_v)
        for rg in range(DR // L):
            d0c = d0_v[pl.ds(rg * L, L)]
            d1c = d1_v[pl.ds(rg * L, L)]
            dall = d0c + d1c + 1e-16
            for rl in range(L):
                r = rg * L + rl
                d = dall[rl]
                for c8 in range(HID // L):
                    cs = pl.ds(c8 * L, L)
                    v = (o0_v[r, cs] + o1_v[r, cs]) / d + bcs[c8] + sk_v[r, cs]
                    xb_v[r, cs] = jnp.where(v > 0.0, v, jnp.exp(v) - 1.0)
        pltpu.sync_copy(xb_v, x_o.at[pl.ds(rb, DR), :])
        return 0

    lax.fori_loop(0, RPD, row_chunk, 0)


def _mm_body(nw):
    def body(x_ref, p_ref, wx_ref, wp_ref, b_ref, *out_refs):
        x = x_ref[...]
        p = p_ref[...]
        for i in range(nw):
            acc = jnp.dot(x, wx_ref[i], preferred_element_type=_f32)
            acc = acc + jnp.dot(p, wp_ref[i], preferred_element_type=_f32)
            out_refs[i][...] = acc + b_ref[i, 0:1, :]
    return body


def _lin(x, posp, wxs, wps, bs, nw):
    bn = 2048
    return pl.pallas_call(
        _mm_body(nw),
        grid=(NP // bn,),
        in_specs=[
            pl.BlockSpec((bn, HID), lambda i: (i, 0)),
            pl.BlockSpec((bn, HID), lambda i: (i, 0)),
            pl.BlockSpec((nw, HID, HID), lambda i: (0, 0, 0)),
            pl.BlockSpec((nw, HID, HID), lambda i: (0, 0, 0)),
            pl.BlockSpec((nw, 8, HID), lambda i: (0, 0, 0)),
        ],
        out_specs=[pl.BlockSpec((bn, HID), lambda i: (i, 0))] * nw,
        out_shape=[jax.ShapeDtypeStruct((NP, HID), _f32)] * nw,
    )(x, posp, wxs, wps, bs)


def _pad_posw(w):
    return jnp.zeros((HID, HID), _f32).at[:3, :].set(w[:, HID:].T)


def _pad_bias(b):
    return jnp.zeros((8, HID), _f32).at[0, :b.shape[0]].set(b)


def kernel(latent, pos, edge_attr, params, edge_index):
    del edge_attr
    prm = params
    src_p = jnp.pad(edge_index[0], (0, EP - E))
    dst_p = jnp.pad(edge_index[1], (0, EP - E))
    lat_p = jnp.pad(latent.astype(_f32), ((0, NP - N), (0, 0)))
    pos_p = jnp.pad(pos.astype(_f32), ((0, NP - N), (0, HID - 3)))
    px = jnp.pad(pos[:, 0], (0, NP - N))
    py = jnp.pad(pos[:, 1], (0, NP - N))
    pz = jnp.pad(pos[:, 2], (0, NP - N))

    zero_w = jnp.zeros((1, HID, HID), _f32)
    x = _lin(lat_p, pos_p, prm['W0'].T[None], zero_w,
             _pad_bias(prm['b0'])[None], 1)[0]

    layers = [(prm['gat0'], prm['W1'], prm['b1']),
              (prm['gat1'], prm['W2'], prm['b2']),
              (prm['gat2'], prm['W3'], prm['b3'])]
    for gat, wk, bk in layers:
        wxs = jnp.stack([gat['Wl'][:, :HID].T, gat['Wr'][:, :HID].T,
                         wk[:, :HID].T])
        wps = jnp.stack([_pad_posw(gat['Wl']), _pad_posw(gat['Wr']),
                         _pad_posw(wk)])
        bss = jnp.stack([_pad_bias(gat['bl']), _pad_bias(gat['br']),
                         _pad_bias(bk)])
        xl, xr, skip = _lin(x, pos_p, wxs, wps, bss, 3)
        alpha, pmax = _sc_attn(xl, xr, px, py, pz, src_p, dst_p,
                               gat['att'], gat['We'][:, 0],
                               gat['We'][:, 1], gat['We'][:, 2])
        outp, denp = _sc_aggr(xl, src_p, dst_p, alpha, pmax)
        x = _sc_norm(outp, denp, gat['bias'], skip)

    w4x = jnp.zeros((HID, HID), _f32).at[:, :OUT].set(prm['W4'][:, :HID].T)
    w4p = jnp.zeros((HID, HID), _f32).at[:3, :OUT].set(prm['W4'][:, HID:].T)
    out = _lin(x, pos_p, w4x[None], w4p[None], _pad_bias(prm['b4'])[None],
               1)[0]
    return out[:N, :OUT]
